# baseline (speedup 1.0000x reference)
import functools
import math

import jax
import jax.numpy as jnp
from jax import lax
from jax.experimental import pallas as pl
from jax.experimental.pallas import tpu as pltpu
from jax.experimental.pallas import tpu_sc as plsc

D_MODEL = 64
SCALE = float(math.sqrt(D_MODEL))

NUM_CORES = 2
NUM_SUBCORES = 16
NUM_WORKERS = NUM_CORES * NUM_SUBCORES

BBLK = 128


def _make_sc_lookup(b: int, t: int, d: int, vocab: int):
    assert b == NUM_WORKERS * BBLK and d == 64

    mesh = plsc.VectorSubcoreMesh(core_axis_name="c", subcore_axis_name="s")

    @functools.partial(
        pl.kernel,
        out_type=jax.ShapeDtypeStruct((t, d, b), jnp.float32),
        mesh=mesh,
        scratch_types=[
            pltpu.VMEM((t, BBLK), jnp.int32),
            pltpu.VMEM((t, BBLK), jnp.int32),
            pltpu.VMEM((BBLK, 128), jnp.float32),
            pltpu.VMEM((d, BBLK), jnp.float32),
            pltpu.SemaphoreType.DMA,
        ],
        compiler_params=pltpu.CompilerParams(needs_layout_passes=False),
    )
    def lookup(xt_hbm, lutp_hbm, out_hbm, idx_v, idx2_v, rows_v, obuf_v, sem):
        w = lax.axis_index("s") * NUM_CORES + lax.axis_index("c")
        pltpu.sync_copy(xt_hbm.at[:, pl.ds(w * BBLK, BBLK)], idx_v)

        def pair_body(i, carry):
            for jc in range(BBLK // 16):
                sl = idx_v[i, pl.ds(jc * 16, 16)]
                idx2_v[i, pl.ds(jc * 16, 16)] = sl >> 1
            return carry

        lax.fori_loop(0, t, pair_body, 0, unroll=2)

        def t_body(g, carry):
            pltpu.async_copy(lutp_hbm.at[idx2_v.at[g]], rows_v, sem).wait()

            for jc in range(BBLK // 16):
                lane = jax.lax.iota(jnp.int32, 16)
                j_idx = lane + (jc * 16)
                par = idx_v[g, pl.ds(jc * 16, 16)]
                pb = (par & 1) * d

                def d_body(d_, carry2):
                    col = pb + d_
                    v = plsc.load_gather(rows_v, [j_idx, col])
                    obuf_v[d_, pl.ds(jc * 16, 16)] = v * SCALE
                    return carry2

                lax.fori_loop(0, d, d_body, 0, unroll=4)

            pltpu.sync_copy(obuf_v, out_hbm.at[g, :, pl.ds(w * BBLK, BBLK)])
            return carry

        lax.fori_loop(0, t, t_body, 0)

    return lookup


def kernel(x, lut):
    b, t = x.shape
    vocab, d = lut.shape
    xt = x.T.astype(jnp.int32)
    lutp = lut.reshape(vocab // 2, 2 * d)
    out = _make_sc_lookup(b, t, d, vocab)(xt, lutp)
    return jnp.transpose(out, (2, 0, 1))


# speedup vs baseline: 1.1482x; 1.1482x over previous
import functools
import math

import jax
import jax.numpy as jnp
from jax import lax
from jax.experimental import pallas as pl
from jax.experimental.pallas import tpu as pltpu
from jax.experimental.pallas import tpu_sc as plsc

D_MODEL = 64
SCALE = float(math.sqrt(D_MODEL))

NUM_CORES = 2
NUM_SUBCORES = 16
NUM_WORKERS = NUM_CORES * NUM_SUBCORES

BBLK = 128
OSTRIDE = 129


def _make_sc_lookup(b: int, t: int, d: int, vocab: int):
    assert b == NUM_WORKERS * BBLK and d == 64

    mesh = plsc.VectorSubcoreMesh(core_axis_name="c", subcore_axis_name="s")

    @functools.partial(
        pl.kernel,
        out_type=jax.ShapeDtypeStruct((t, d, b), jnp.float32),
        mesh=mesh,
        scratch_types=[
            pltpu.VMEM((t, BBLK), jnp.int32),
            pltpu.VMEM((BBLK, 128), jnp.float32),
            pltpu.VMEM((d, OSTRIDE), jnp.float32),
            pltpu.SemaphoreType.DMA,
        ],
        compiler_params=pltpu.CompilerParams(needs_layout_passes=False),
    )
    def lookup(xt_hbm, lutp_hbm, out_hbm, idx_v, rows_v, obuf_v, sem):
        w = lax.axis_index("s") * NUM_CORES + lax.axis_index("c")
        pltpu.sync_copy(xt_hbm.at[:, pl.ds(w * BBLK, BBLK)], idx_v)

        lane = jax.lax.iota(jnp.int32, 16)
        kc_rows = [lane + kc * 16 for kc in range(d // 16)]

        def t_body(g, carry):
            pltpu.async_copy(lutp_hbm.at[idx_v.at[g]], rows_v, sem).wait()

            def j_body(j, carry2):
                jv = jnp.full((16,), 0, jnp.int32) + j
                for kc in range(d // 16):
                    v = rows_v[j, pl.ds(kc * 16, 16)] * SCALE
                    plsc.store_scatter(obuf_v, [kc_rows[kc], jv], v)
                return carry2

            lax.fori_loop(0, BBLK, j_body, 0, unroll=2)

            pltpu.sync_copy(obuf_v.at[:, pl.ds(0, BBLK)],
                            out_hbm.at[g, :, pl.ds(w * BBLK, BBLK)])
            return carry

        lax.fori_loop(0, t, t_body, 0)

    return lookup


def kernel(x, lut):
    b, t = x.shape
    vocab, d = lut.shape
    xt = x.T.astype(jnp.int32)
    lutp = jnp.pad(lut, ((0, 0), (0, 128 - d)))
    out = _make_sc_lookup(b, t, d, vocab)(xt, lutp)
    return jnp.transpose(out, (2, 0, 1))


# speedup vs baseline: 1.4443x; 1.2579x over previous
import functools
import math

import jax
import jax.numpy as jnp
from jax import lax
from jax.experimental import pallas as pl
from jax.experimental.pallas import tpu as pltpu
from jax.experimental.pallas import tpu_sc as plsc

D_MODEL = 64
SCALE = float(math.sqrt(D_MODEL))

NUM_CORES = 2
NUM_SUBCORES = 16
NUM_WORKERS = NUM_CORES * NUM_SUBCORES

BBLK = 128
OSTRIDE = 129


def _make_sc_lookup(b: int, t: int, d: int, vocab: int):
    assert b == NUM_WORKERS * BBLK and d == 64

    mesh = plsc.VectorSubcoreMesh(core_axis_name="c", subcore_axis_name="s")

    @functools.partial(
        pl.kernel,
        out_type=jax.ShapeDtypeStruct((t, d, b), jnp.float32),
        mesh=mesh,
        scratch_types=[
            pltpu.VMEM((t, BBLK), jnp.int32),
            pltpu.VMEM((BBLK, 128), jnp.float32),
            pltpu.VMEM((d, OSTRIDE), jnp.float32),
            pltpu.SemaphoreType.DMA,
        ],
        compiler_params=pltpu.CompilerParams(needs_layout_passes=False),
    )
    def lookup(xt_hbm, lutp_hbm, out_hbm, idx_v, rows_v, obuf_v, sem):
        w = lax.axis_index("s") * NUM_CORES + lax.axis_index("c")
        pltpu.sync_copy(xt_hbm.at[:, pl.ds(w * BBLK, BBLK)], idx_v)

        lane = jax.lax.iota(jnp.int32, 16)
        kc_rows = [lane + kc * 16 for kc in range(d // 16)]

        def t_body(g, carry):
            pltpu.async_copy(lutp_hbm.at[idx_v.at[g]], rows_v, sem).wait()

            @plsc.parallel_loop(0, BBLK, 1, unroll=4)
            def j_body(j):
                jv = jnp.full((16,), 0, jnp.int32) + j
                for kc in range(d // 16):
                    v = rows_v[j, pl.ds(kc * 16, 16)] * SCALE
                    plsc.store_scatter(obuf_v, [kc_rows[kc], jv], v)

            pltpu.sync_copy(obuf_v.at[:, pl.ds(0, BBLK)],
                            out_hbm.at[g, :, pl.ds(w * BBLK, BBLK)])
            return carry

        lax.fori_loop(0, t, t_body, 0)

    return lookup


def kernel(x, lut):
    b, t = x.shape
    vocab, d = lut.shape
    xt = x.T.astype(jnp.int32)
    lutp = jnp.pad(lut, ((0, 0), (0, 128 - d)))
    out = _make_sc_lookup(b, t, d, vocab)(xt, lutp)
    return jnp.transpose(out, (2, 0, 1))


# speedup vs baseline: 1.8819x; 1.3030x over previous
import functools
import math

import jax
import jax.numpy as jnp
from jax import lax
from jax.experimental import pallas as pl
from jax.experimental.pallas import tpu as pltpu
from jax.experimental.pallas import tpu_sc as plsc

D_MODEL = 64
SCALE = float(math.sqrt(D_MODEL))

NUM_CORES = 2
NUM_SUBCORES = 16
NUM_WORKERS = NUM_CORES * NUM_SUBCORES

BBLK = 128
OSTRIDE = 129


def _make_sc_lookup(b: int, t: int, d: int, vocab: int):
    assert b == NUM_WORKERS * BBLK and d == 64 and t % 2 == 0

    mesh = plsc.VectorSubcoreMesh(core_axis_name="c", subcore_axis_name="s")

    @functools.partial(
        pl.kernel,
        out_type=jax.ShapeDtypeStruct((t, d, b), jnp.float32),
        mesh=mesh,
        scratch_types=[
            pltpu.VMEM((t, BBLK), jnp.int32),
            pltpu.VMEM((2, BBLK, 128), jnp.float32),
            pltpu.VMEM((2, d, OSTRIDE), jnp.float32),
            pltpu.SemaphoreType.DMA,
            pltpu.SemaphoreType.DMA,
            pltpu.SemaphoreType.DMA,
            pltpu.SemaphoreType.DMA,
        ],
        compiler_params=pltpu.CompilerParams(needs_layout_passes=False),
    )
    def lookup(xt_hbm, lutp_hbm, out_hbm, idx_v, rows_v, obuf_v,
               gs0, gs1, os0, os1):
        gsems = (gs0, gs1)
        osems = (os0, os1)
        w = lax.axis_index("s") * NUM_CORES + lax.axis_index("c")
        ob = pl.ds(w * BBLK, BBLK)
        pltpu.sync_copy(xt_hbm.at[:, ob], idx_v)

        lane = jax.lax.iota(jnp.int32, 16)
        kc_rows = [lane + kc * 16 for kc in range(d // 16)]

        for par in range(2):
            pltpu.async_copy(lutp_hbm.at[idx_v.at[par]],
                             rows_v.at[par], gsems[par])

        def gg_body(gg, carry):
            for par in range(2):
                g = gg * 2 + par
                pltpu.make_async_copy(lutp_hbm.at[idx_v.at[g]],
                                      rows_v.at[par], gsems[par]).wait()

                @pl.when(gg > 0)
                def _wait_out():
                    pltpu.make_async_copy(
                        obuf_v.at[par, :, pl.ds(0, BBLK)],
                        out_hbm.at[g - 2, :, ob], osems[par]).wait()

                @plsc.parallel_loop(0, BBLK, 1, unroll=4)
                def j_body(j):
                    jv = jnp.full((16,), 0, jnp.int32) + j
                    for kc in range(d // 16):
                        v = rows_v[par, j, pl.ds(kc * 16, 16)] * SCALE
                        plsc.store_scatter(obuf_v.at[par],
                                           [kc_rows[kc], jv], v)

                pltpu.async_copy(obuf_v.at[par, :, pl.ds(0, BBLK)],
                                 out_hbm.at[g, :, ob], osems[par])

                @pl.when(gg < t // 2 - 1)
                def _next_gather():
                    pltpu.async_copy(lutp_hbm.at[idx_v.at[g + 2]],
                                     rows_v.at[par], gsems[par])
            return carry

        lax.fori_loop(0, t // 2, gg_body, 0)

        for par, gl in ((0, t - 2), (1, t - 1)):
            pltpu.make_async_copy(obuf_v.at[par, :, pl.ds(0, BBLK)],
                                  out_hbm.at[gl, :, ob], osems[par]).wait()

    return lookup


def kernel(x, lut):
    b, t = x.shape
    vocab, d = lut.shape
    xt = x.T.astype(jnp.int32)
    lutp = jnp.pad(lut, ((0, 0), (0, 128 - d)))
    out = _make_sc_lookup(b, t, d, vocab)(xt, lutp)
    return jnp.transpose(out, (2, 0, 1))
